# Initial kernel scaffold; baseline (speedup 1.0000x reference)
import jax
import jax.numpy as jnp
from jax.experimental import pallas as pl


def kernel(x, weight):
    raise NotImplementedError("write your pallas kernel here")


# speedup vs baseline: 1.4938x; 1.4938x over previous
import functools

import jax
import jax.numpy as jnp
from jax import lax
from jax.experimental import pallas as pl
from jax.experimental.pallas import tpu as pltpu
from jax.experimental.pallas import tpu_sc as plsc

VOCAB_SIZE = 1000000
EMBED_DIM = 32
BATCH = 4096
POS = 200

NTOT = BATCH * POS
CHUNK = 128
NROWS = NTOT // CHUNK
NUM_WORKERS = 32
ROWS_PER_W = NROWS // NUM_WORKERS
GROUP = 20
GROUPS = ROWS_PER_W // GROUP

_mesh = plsc.VectorSubcoreMesh(core_axis_name="c", subcore_axis_name="s")


@functools.partial(
    pl.kernel,
    mesh=_mesh,
    out_type=jax.ShapeDtypeStruct((NROWS, CHUNK, EMBED_DIM), jnp.float32),
    scratch_types=[
        pltpu.VMEM((ROWS_PER_W, CHUNK), jnp.int32),
        pltpu.VMEM((GROUP, CHUNK, EMBED_DIM), jnp.float32),
        pltpu.SemaphoreType.DMA,
    ],
    compiler_params=pltpu.CompilerParams(use_tc_tiling_on_sc=False),
)
def _embed_gather(idx_hbm, table_hbm, out_hbm, idx_v, rows_v, sem):
    wid = lax.axis_index("s") * 2 + lax.axis_index("c")
    rbase = wid * ROWS_PER_W
    pltpu.sync_copy(idx_hbm.at[pl.ds(rbase, ROWS_PER_W)], idx_v)

    def body(g, carry):
        descs = []
        for j in range(GROUP):
            d = pltpu.async_copy(
                table_hbm.at[idx_v.at[g * GROUP + j]], rows_v.at[j], sem
            )
            descs.append(d)
        for d in descs:
            d.wait()
        pltpu.sync_copy(rows_v, out_hbm.at[pl.ds(rbase + g * GROUP, GROUP)])
        return carry

    lax.fori_loop(0, GROUPS, body, 0)


def kernel(x, weight):
    idx = x.reshape(NROWS, CHUNK).astype(jnp.int32)
    out = _embed_gather(idx, weight)
    return out.reshape(BATCH, POS, EMBED_DIM)
